# speedup vs baseline: 7.1178x; 7.1178x over previous
import functools

import jax
import jax.numpy as jnp
from jax.experimental import pallas as pl
from jax.experimental.pallas import tpu as pltpu

B, N, M, D = 16, 512, 1024, 256
EPS = 1e-6


def _body(q_ref, k_ref, v_ref, wv_ref, out_ref):
    q = q_ref[0]
    k = k_ref[0]
    logits = jax.lax.dot_general(
        q, k, (((1,), (1,)), ((), ())), preferred_element_type=jnp.float32
    )
    maxv = jnp.max(logits, axis=0, keepdims=True)
    iota_n = jax.lax.broadcasted_iota(jnp.int32, (N, M), 0)
    idx = jnp.min(jnp.where(logits == maxv, iota_n, N), axis=0, keepdims=True)
    onehot = (iota_n == idx).astype(jnp.float32)
    vv = jax.lax.dot_general(
        v_ref[0], wv_ref[...], (((1,), (1,)), ((), ())),
        preferred_element_type=jnp.float32,
    )
    sums = jax.lax.dot_general(
        onehot, vv, (((1,), (0,)), ((), ())), preferred_element_type=jnp.float32
    )
    counts = jnp.sum(onehot, axis=1, keepdims=True)
    out_ref[0] = sums / (jnp.maximum(counts, 1.0) + EPS)


@jax.jit
def kernel(q, k, v, Wv):
    return pl.pallas_call(
        _body,
        grid=(B,),
        in_specs=[
            pl.BlockSpec((1, N, D), lambda b: (b, 0, 0)),
            pl.BlockSpec((1, M, D), lambda b: (b, 0, 0)),
            pl.BlockSpec((1, M, D), lambda b: (b, 0, 0)),
            pl.BlockSpec((D, D), lambda b: (0, 0)),
        ],
        out_specs=pl.BlockSpec((1, N, D), lambda b: (b, 0, 0)),
        out_shape=jax.ShapeDtypeStruct((B, N, D), jnp.float32),
    )(q, k, v, Wv)
